# Initial kernel scaffold; baseline (speedup 1.0000x reference)
import jax
import jax.numpy as jnp
from jax.experimental import pallas as pl


def kernel(molecular_features, W0, b0, W1, b1, W2, b2, W_out, b_out):
    raise NotImplementedError("write your pallas kernel here")


# speedup vs baseline: 2.0162x; 2.0162x over previous
import functools

import jax
import jax.numpy as jnp
from jax.experimental import pallas as pl

_D = 128
_ATOMS = 64
_TILE = 256


def _encoder_kernel(x_ref, w0_ref, b0_ref, w1_ref, b1_ref, w2_ref, b2_ref,
                    wout_ref, bout_ref, o_ref):
    x = x_ref[...].reshape(_TILE * _ATOMS, _D)
    for w_ref, b_ref in ((w0_ref, b0_ref), (w1_ref, b1_ref), (w2_ref, b2_ref)):
        x = jnp.dot(x, w_ref[...], preferred_element_type=jnp.float32)
        x = jnp.maximum(x + b_ref[...], 0.0)
    pooled = jnp.mean(x.reshape(_TILE, _ATOMS, _D), axis=1)
    o_ref[...] = (
        jnp.dot(pooled, wout_ref[...], preferred_element_type=jnp.float32)
        + bout_ref[...]
    )


@jax.jit
def kernel(molecular_features, W0, b0, W1, b1, W2, b2, W_out, b_out):
    n_mol, atoms, d = molecular_features.shape
    hidden = W_out.shape[1]
    grid = (n_mol // _TILE,)

    weight_specs = []
    weight_args = []
    for w, b in ((W0, b0), (W1, b1), (W2, b2), (W_out, b_out)):
        weight_args.append(w)
        weight_args.append(b.reshape(1, -1))
        weight_specs.append(pl.BlockSpec(w.shape, lambda i: (0, 0)))
        weight_specs.append(pl.BlockSpec((1, b.shape[0]), lambda i: (0, 0)))

    return pl.pallas_call(
        _encoder_kernel,
        grid=grid,
        in_specs=[
            pl.BlockSpec((_TILE, atoms, d), lambda i: (i, 0, 0)),
            *weight_specs,
        ],
        out_specs=pl.BlockSpec((_TILE, hidden), lambda i: (i, 0)),
        out_shape=jax.ShapeDtypeStruct((n_mol, hidden), jnp.float32),
    )(molecular_features, *weight_args)
